# baseline (speedup 1.0000x reference)
import functools

import jax
import jax.numpy as jnp
from jax import lax
from jax.experimental import pallas as pl
from jax.experimental.pallas import tpu as pltpu
from jax.experimental.pallas import tpu_sc as plsc

HIDDEN_DIM = 2048
DELTA = 1.0 / HIDDEN_DIM
GAMMA = 0.05
LAMBDA_KL = 0.01
BETA = 0.05

ROWS = 4 * 8192
CHUNK = 2048
GROUP = 128
NGROUPS = HIDDEN_DIM // GROUP

NTILES = 32
RPT = ROWS // NTILES
CROWS = 16
NCHUNKS = RPT // CROWS
NBUF = 2
NV = HIDDEN_DIM // 16


def _ds16(i):
    return pl.ds(pl.multiple_of(i * 16, 16), 16)


def _sc_body(x_hbm, h_hbm, psum_hbm, pcnt_hbm,
             buf, acc, cnt, s_in0, s_in1, s_out0, s_out1, s_res):
    in_sems = (s_in0, s_in1)
    out_sems = (s_out0, s_out1)
    wid = lax.axis_index("s") * 2 + lax.axis_index("c")
    base = wid * RPT

    def zero(i, c):
        acc[_ds16(i)] = jnp.zeros((16,), jnp.float32)
        cnt[_ds16(i)] = jnp.zeros((16,), jnp.float32)
        return c

    lax.fori_loop(0, NV, zero, 0)

    for b in range(NBUF):
        pltpu.make_async_copy(
            x_hbm.at[pl.ds(base + b * CROWS, CROWS)], buf.at[b], in_sems[b]
        ).start()

    def outer(t, carry):
        for b in range(NBUF):
            k = t * NBUF + b
            row0 = base + k * CROWS
            pltpu.make_async_copy(
                x_hbm.at[pl.ds(row0, CROWS)], buf.at[b], in_sems[b]
            ).wait()
            pltpu.make_async_copy(
                buf.at[b], h_hbm.at[pl.ds(row0, CROWS)], out_sems[b]
            ).start()

            def col_block(cb, c2):
                a = acc[_ds16(cb)]
                c = cnt[_ds16(cb)]
                for r in range(CROWS):
                    v = buf[b, r, _ds16(cb)]
                    m = v > 0.0
                    a = jnp.where(m, a + v, a)
                    c = jnp.where(m, c + 1.0, c)
                acc[_ds16(cb)] = a
                cnt[_ds16(cb)] = c
                return c2

            lax.fori_loop(0, NV, col_block, 0)

            @pl.when(k + NBUF < NCHUNKS)
            def _():
                nrow0 = base + (k + NBUF) * CROWS
                pltpu.make_async_copy(
                    buf.at[b], h_hbm.at[pl.ds(row0, CROWS)], out_sems[b]
                ).wait()
                pltpu.make_async_copy(
                    x_hbm.at[pl.ds(nrow0, CROWS)], buf.at[b], in_sems[b]
                ).start()
        return carry

    lax.fori_loop(0, NCHUNKS // NBUF, outer, 0)

    for b in range(NBUF):
        row0 = base + (NCHUNKS - NBUF + b) * CROWS
        pltpu.make_async_copy(
            buf.at[b], h_hbm.at[pl.ds(row0, CROWS)], out_sems[b]
        ).wait()

    pltpu.make_async_copy(acc, psum_hbm.at[wid], s_res).start()
    pltpu.make_async_copy(acc, psum_hbm.at[wid], s_res).wait()
    pltpu.make_async_copy(cnt, pcnt_hbm.at[wid], s_res).start()
    pltpu.make_async_copy(cnt, pcnt_hbm.at[wid], s_res).wait()


def _sc_stream(xf):
    mesh = plsc.VectorSubcoreMesh(core_axis_name="c", subcore_axis_name="s")
    f = functools.partial(
        pl.kernel,
        out_type=[
            jax.ShapeDtypeStruct((ROWS, HIDDEN_DIM), jnp.float32),
            jax.ShapeDtypeStruct((NTILES, HIDDEN_DIM), jnp.float32),
            jax.ShapeDtypeStruct((NTILES, HIDDEN_DIM), jnp.float32),
        ],
        mesh=mesh,
        scratch_types=[
            pltpu.VMEM((NBUF, CROWS, HIDDEN_DIM), jnp.float32),
            pltpu.VMEM((HIDDEN_DIM,), jnp.float32),
            pltpu.VMEM((HIDDEN_DIM,), jnp.float32),
        ] + [pltpu.SemaphoreType.DMA] * 5,
    )(_sc_body)
    return f(xf)


def _epi_body(ps_ref, pc_ref, e_ref, n_ref,
              ne_ref, aux_ref, msk_ref, val_ref, gcnt_ref):
    colsum = jnp.sum(ps_ref[...], axis=0, keepdims=True)
    colmean = colsum * (1.0 / ROWS)
    e = e_ref[...]
    ne = e + DELTA + n_ref[...] - GAMMA * colmean
    rho = jnp.sum(pc_ref[...]) * (1.0 / (ROWS * HIDDEN_DIM))
    rho = jnp.clip(rho, 1e-05, 1.0 - 1e-05)
    kl = LAMBDA_KL * (rho * jnp.log(rho / BETA)
                      + (1.0 - rho) * jnp.log((1.0 - rho) / (1.0 - BETA)))
    high = ne > 1.0
    low = ne < -1.0
    pen = (0.01 * jnp.sum(jnp.where(high, jnp.abs(ne) - 1.0, 0.0))
           + 0.01 * jnp.sum(jnp.where(low, jnp.abs(ne) - 1.0, 0.0)))
    aux_ref[0, 0] = kl + pen
    fire = ne >= 2.0
    shut = ne <= -2.0
    ne_ref[...] = jnp.where(shut, -2.0, ne)
    m = jnp.logical_or(fire, shut)
    mi = m.astype(jnp.int32)
    msk_ref[...] = m.astype(jnp.float32)
    val_ref[...] = jnp.where(shut, e + 2.0, 2.0)
    for k in range(NGROUPS):
        gcnt_ref[0, k] = jnp.sum(mi[0, k * GROUP:(k + 1) * GROUP])


def _fix_body(h_in_ref, msk_ref, val_ref, gcnt_ref, h_ref, buf, sem):
    del h_in_ref
    for g in range(NGROUPS):
        @pl.when(gcnt_ref[0, g] > 0)
        def _():
            mg = msk_ref[0:1, g * GROUP:(g + 1) * GROUP] > 0.5
            vg = val_ref[0:1, g * GROUP:(g + 1) * GROUP]

            def per_chunk(r, c):
                stripe = h_ref.at[pl.ds(r * CHUNK, CHUNK),
                                  pl.ds(g * GROUP, GROUP)]
                pltpu.make_async_copy(stripe, buf, sem).start()
                pltpu.make_async_copy(stripe, buf, sem).wait()
                buf[...] = jnp.where(mg, vg, buf[...])
                pltpu.make_async_copy(buf, stripe, sem).start()
                pltpu.make_async_copy(buf, stripe, sem).wait()
                return c

            lax.fori_loop(0, ROWS // CHUNK, per_chunk, 0)


@jax.jit
def kernel(x, energy, noise):
    xf = x.reshape(ROWS, HIDDEN_DIM)
    e2 = energy.reshape(1, HIDDEN_DIM)
    n2 = noise.reshape(1, HIDDEN_DIM)

    h0, psum, pcnt = _sc_stream(xf)

    ne, aux, msk, val, gcnt = pl.pallas_call(
        _epi_body,
        in_specs=[
            pl.BlockSpec(memory_space=pltpu.VMEM),
            pl.BlockSpec(memory_space=pltpu.VMEM),
            pl.BlockSpec(memory_space=pltpu.VMEM),
            pl.BlockSpec(memory_space=pltpu.VMEM),
        ],
        out_specs=[
            pl.BlockSpec(memory_space=pltpu.VMEM),
            pl.BlockSpec(memory_space=pltpu.SMEM),
            pl.BlockSpec(memory_space=pltpu.VMEM),
            pl.BlockSpec(memory_space=pltpu.VMEM),
            pl.BlockSpec(memory_space=pltpu.SMEM),
        ],
        out_shape=[
            jax.ShapeDtypeStruct((1, HIDDEN_DIM), jnp.float32),
            jax.ShapeDtypeStruct((1, 1), jnp.float32),
            jax.ShapeDtypeStruct((1, HIDDEN_DIM), jnp.float32),
            jax.ShapeDtypeStruct((1, HIDDEN_DIM), jnp.float32),
            jax.ShapeDtypeStruct((1, NGROUPS), jnp.int32),
        ],
    )(psum, pcnt, e2, n2)

    h = pl.pallas_call(
        _fix_body,
        in_specs=[
            pl.BlockSpec(memory_space=pl.ANY),
            pl.BlockSpec(memory_space=pltpu.VMEM),
            pl.BlockSpec(memory_space=pltpu.VMEM),
            pl.BlockSpec(memory_space=pltpu.SMEM),
        ],
        out_specs=pl.BlockSpec(memory_space=pl.ANY),
        out_shape=jax.ShapeDtypeStruct((ROWS, HIDDEN_DIM), jnp.float32),
        scratch_shapes=[
            pltpu.VMEM((CHUNK, GROUP), jnp.float32),
            pltpu.SemaphoreType.DMA,
        ],
        input_output_aliases={0: 0},
    )(h0, msk, val, gcnt)

    return (h.reshape(x.shape), ne.reshape(HIDDEN_DIM), aux[0, 0])


# speedup vs baseline: 1.7234x; 1.7234x over previous
import functools

import jax
import jax.numpy as jnp
from jax import lax
from jax.experimental import pallas as pl
from jax.experimental.pallas import tpu as pltpu

HIDDEN_DIM = 2048
DELTA = 1.0 / HIDDEN_DIM
GAMMA = 0.05
LAMBDA_KL = 0.01
BETA = 0.05

ROWS = 4 * 8192
BLOCK_ROWS = 1024
NSTEPS = ROWS // BLOCK_ROWS
CHUNK = 2048
GROUP = 128
NGROUPS = HIDDEN_DIM // GROUP


def _main_body(x_ref, e_ref, n_ref,
               h_ref, ne_ref, aux_ref, msk_ref, val_ref, gcnt_ref,
               acc_ref, cnt_ref):
    i = pl.program_id(0)
    xb = x_ref[...]
    h_ref[...] = xb
    relu = jnp.maximum(xb, 0.0)
    psum = jnp.sum(relu, axis=0, keepdims=True)
    pcnt = jnp.sum((xb > 0.0).astype(jnp.float32), axis=0, keepdims=True)

    @pl.when(i == 0)
    def _():
        acc_ref[...] = psum
        cnt_ref[...] = pcnt

    @pl.when(i > 0)
    def _():
        acc_ref[...] += psum
        cnt_ref[...] += pcnt

    @pl.when(i == NSTEPS - 1)
    def _():
        colmean = acc_ref[...] * (1.0 / ROWS)
        e = e_ref[...]
        ne = e + DELTA + n_ref[...] - GAMMA * colmean
        rho = jnp.sum(cnt_ref[...]) * (1.0 / (ROWS * HIDDEN_DIM))
        rho = jnp.clip(rho, 1e-05, 1.0 - 1e-05)
        kl = LAMBDA_KL * (rho * jnp.log(rho / BETA)
                          + (1.0 - rho) * jnp.log((1.0 - rho) / (1.0 - BETA)))
        high = ne > 1.0
        low = ne < -1.0
        pen = (0.01 * jnp.sum(jnp.where(high, jnp.abs(ne) - 1.0, 0.0))
               + 0.01 * jnp.sum(jnp.where(low, jnp.abs(ne) - 1.0, 0.0)))
        aux_ref[0, 0] = kl + pen
        fire = ne >= 2.0
        shut = ne <= -2.0
        ne_ref[...] = jnp.where(shut, -2.0, ne)
        m = jnp.logical_or(fire, shut)
        mi = m.astype(jnp.int32)
        msk_ref[...] = m.astype(jnp.float32)
        val_ref[...] = jnp.where(shut, e + 2.0, 2.0)
        for k in range(NGROUPS):
            gcnt_ref[0, k] = jnp.sum(mi[0, k * GROUP:(k + 1) * GROUP])


def _fix_body(h_in_ref, msk_ref, val_ref, gcnt_ref, h_ref, buf, sem):
    del h_in_ref
    for g in range(NGROUPS):
        @pl.when(gcnt_ref[0, g] > 0)
        def _():
            mg = msk_ref[0:1, g * GROUP:(g + 1) * GROUP] > 0.5
            vg = val_ref[0:1, g * GROUP:(g + 1) * GROUP]

            def per_chunk(r, c):
                stripe = h_ref.at[pl.ds(r * CHUNK, CHUNK),
                                  pl.ds(g * GROUP, GROUP)]
                pltpu.make_async_copy(stripe, buf, sem).start()
                pltpu.make_async_copy(stripe, buf, sem).wait()
                buf[...] = jnp.where(mg, vg, buf[...])
                pltpu.make_async_copy(buf, stripe, sem).start()
                pltpu.make_async_copy(buf, stripe, sem).wait()
                return c

            lax.fori_loop(0, ROWS // CHUNK, per_chunk, 0)


@jax.jit
def kernel(x, energy, noise):
    xf = x.reshape(ROWS, HIDDEN_DIM)
    e2 = energy.reshape(1, HIDDEN_DIM)
    n2 = noise.reshape(1, HIDDEN_DIM)

    h0, ne, aux, msk, val, gcnt = pl.pallas_call(
        _main_body,
        grid=(NSTEPS,),
        in_specs=[
            pl.BlockSpec((BLOCK_ROWS, HIDDEN_DIM), lambda i: (i, 0)),
            pl.BlockSpec((1, HIDDEN_DIM), lambda i: (0, 0)),
            pl.BlockSpec((1, HIDDEN_DIM), lambda i: (0, 0)),
        ],
        out_specs=[
            pl.BlockSpec((BLOCK_ROWS, HIDDEN_DIM), lambda i: (i, 0)),
            pl.BlockSpec((1, HIDDEN_DIM), lambda i: (0, 0)),
            pl.BlockSpec((1, 1), lambda i: (0, 0), memory_space=pltpu.SMEM),
            pl.BlockSpec((1, HIDDEN_DIM), lambda i: (0, 0)),
            pl.BlockSpec((1, HIDDEN_DIM), lambda i: (0, 0)),
            pl.BlockSpec((1, NGROUPS), lambda i: (0, 0), memory_space=pltpu.SMEM),
        ],
        out_shape=[
            jax.ShapeDtypeStruct((ROWS, HIDDEN_DIM), jnp.float32),
            jax.ShapeDtypeStruct((1, HIDDEN_DIM), jnp.float32),
            jax.ShapeDtypeStruct((1, 1), jnp.float32),
            jax.ShapeDtypeStruct((1, HIDDEN_DIM), jnp.float32),
            jax.ShapeDtypeStruct((1, HIDDEN_DIM), jnp.float32),
            jax.ShapeDtypeStruct((1, NGROUPS), jnp.int32),
        ],
        scratch_shapes=[
            pltpu.VMEM((1, HIDDEN_DIM), jnp.float32),
            pltpu.VMEM((1, HIDDEN_DIM), jnp.float32),
        ],
        compiler_params=pltpu.CompilerParams(
            dimension_semantics=("arbitrary",),
        ),
    )(xf, e2, n2)

    h = pl.pallas_call(
        _fix_body,
        in_specs=[
            pl.BlockSpec(memory_space=pl.ANY),
            pl.BlockSpec(memory_space=pltpu.VMEM),
            pl.BlockSpec(memory_space=pltpu.VMEM),
            pl.BlockSpec(memory_space=pltpu.SMEM),
        ],
        out_specs=pl.BlockSpec(memory_space=pl.ANY),
        out_shape=jax.ShapeDtypeStruct((ROWS, HIDDEN_DIM), jnp.float32),
        scratch_shapes=[
            pltpu.VMEM((CHUNK, GROUP), jnp.float32),
            pltpu.SemaphoreType.DMA,
        ],
        input_output_aliases={0: 0},
    )(h0, msk, val, gcnt)

    return (h.reshape(x.shape), ne.reshape(HIDDEN_DIM), aux[0, 0])
